# baseline (speedup 1.0000x reference)
import jax
import jax.numpy as jnp
from jax import lax
from jax.experimental import pallas as pl
from jax.experimental.pallas import tpu as pltpu

LN_EPS = 1e-12


def _pick_tile(slen, max_tile=512):
    tile = slen
    for cand in range(8, min(max_tile, slen) + 1, 8):
        if slen % cand == 0:
            tile = cand
    return tile


def _make_kernel(nt, half, tile):
    def kern(ids_ref,
             word_hbm,
             pos_ref,
             lang_ref,
             gb_ref,
             out_ref,
             wrows,
             sems):
        c = pl.program_id(0)
        j = pl.program_id(1)
        nj = pl.num_programs(1)
        vocab = word_hbm.shape[0]
        slot = j % 2

        def issue(jj, s):
            b = c * half + (jj // nt if nt > 1 else jj)
            tok0 = (jj % nt) * tile if nt > 1 else 0
            dst = wrows.at[s]
            sem = sems.at[s]

            def one(i):
                row = ids_ref[b, tok0 + i]
                row = jnp.clip(row, 0, vocab - 1)
                pltpu.make_async_copy(word_hbm.at[row], dst.at[i], sem).start()

            @pl.loop(0, tile // 8)
            def _(k):
                base = k * 8
                for u in range(8):
                    one(base + u)

        @pl.when(j == 0)
        def _():
            issue(j, slot)

        pltpu.make_async_copy(word_hbm.at[pl.ds(0, tile)], wrows.at[slot],
                              sems.at[slot]).wait()

        @pl.when(j + 1 < nj)
        def _():
            issue(j + 1, 1 - slot)

        x = wrows[slot] + pos_ref[...] + lang_ref[...]
        mean = jnp.mean(x, axis=-1, keepdims=True)
        var = jnp.mean(jnp.square(x - mean), axis=-1, keepdims=True)
        x_hat = (x - mean) * lax.rsqrt(var + LN_EPS)
        gb = gb_ref[...]
        out_ref[0] = x_hat * gb[0:1, :] + gb[1:2, :]

    return kern


def kernel(input_ids, word_embeddings, position_embeddings, lang_embeddings,
           ln_gamma, ln_beta):
    bs, slen = input_ids.shape
    vocab, hidden = word_embeddings.shape

    input_ids = input_ids.astype(jnp.int32)
    lang_row = lang_embeddings[2].astype(jnp.float32)[None, :]
    gb = jnp.stack([ln_gamma.astype(jnp.float32), ln_beta.astype(jnp.float32)])
    pos_slab = position_embeddings[:slen].astype(jnp.float32)

    tile = _pick_tile(slen)
    nt = slen // tile
    ncores = 2 if bs % 2 == 0 else 1
    half = bs // ncores
    nj = half * nt
    assert vocab >= tile

    if nt > 1:
        pos_idx = lambda c, j, ids: (j % nt, 0)
        out_idx = lambda c, j, ids: (c * half + j // nt, j % nt, 0)
    else:
        pos_idx = lambda c, j, ids: (0, 0)
        out_idx = lambda c, j, ids: (c * half + j, 0, 0)

    grid_spec = pltpu.PrefetchScalarGridSpec(
        num_scalar_prefetch=1,
        grid=(ncores, nj),
        in_specs=[
            pl.BlockSpec(memory_space=pl.ANY),
            pl.BlockSpec((tile, hidden), pos_idx),
            pl.BlockSpec((1, hidden), lambda c, j, ids: (0, 0)),
            pl.BlockSpec((2, hidden), lambda c, j, ids: (0, 0)),
        ],
        out_specs=pl.BlockSpec((1, tile, hidden), out_idx),
        scratch_shapes=[
            pltpu.VMEM((2, tile, hidden), jnp.float32),
            pltpu.SemaphoreType.DMA((2,)),
        ],
    )
    return pl.pallas_call(
        _make_kernel(nt, half, tile),
        out_shape=jax.ShapeDtypeStruct((bs, slen, hidden), jnp.float32),
        grid_spec=grid_spec,
        compiler_params=pltpu.CompilerParams(
            dimension_semantics=("parallel", "arbitrary"),
            disable_bounds_checks=True,
        ),
    )(input_ids, word_embeddings, pos_slab, lang_row, gb)
